# baseline (speedup 1.0000x reference)
import dataclasses

import jax
import jax.numpy as jnp
from jax import lax
from jax.experimental import pallas as pl
from jax.experimental.pallas import tpu as pltpu
from jax.experimental.pallas import tpu_sc as plsc

B = 4096
NUM_NUMERIC = 13
NUM_FIELDS = 26
VOCAB = 100000
EMB = 32
EPS = 1e-5

NC, NS = 2, 16
NW = NC * NS
CHUNK = 128
BCHUNKS = B // CHUNK
TOTAL_CHUNKS = NUM_FIELDS * BCHUNKS
CHUNKS_PER_W = TOTAL_CHUNKS // NW
IDX_PER_W = CHUNKS_PER_W * CHUNK

PACK_C = 1280
PACK_NB = 20
QUART = PACK_C * PACK_NB
VROWS = QUART
LAST_COL_BLOCK = (VOCAB - 1) // PACK_C


def _pack_body(t0_ref, t1_ref, t2_ref, t3_ref, o_ref):
    c_id = pl.program_id(1)
    parts = []
    for m, t_ref in enumerate((t0_ref, t1_ref, t2_ref, t3_ref)):
        t = t_ref[0]
        if m == 3:
            src0 = jnp.minimum(m * PACK_NB + c_id, LAST_COL_BLOCK) * PACK_C
            src = src0 + lax.broadcasted_iota(jnp.int32, (EMB, PACK_C), 1)
            t = jnp.where(src < VOCAB, t, 0.0)
        parts.append(t.T)
    o_ref[0] = jnp.concatenate(parts, axis=1)


def _pack_tables(tab_t):
    def spec(m):
        return pl.BlockSpec(
            (1, EMB, PACK_C),
            lambda f, c, m=m: (f, 0, jnp.minimum(m * PACK_NB + c,
                                                 LAST_COL_BLOCK)))
    return pl.pallas_call(
        _pack_body,
        grid=(NUM_FIELDS, PACK_NB),
        in_specs=[spec(0), spec(1), spec(2), spec(3)],
        out_specs=pl.BlockSpec((1, PACK_C, 128), lambda f, c: (f, c, 0)),
        out_shape=jax.ShapeDtypeStruct((NUM_FIELDS, VROWS, 128), jnp.float32),
    )(tab_t, tab_t, tab_t, tab_t)


def _sc_gather(tab4, idx4, m32):
    mesh = plsc.VectorSubcoreMesh(core_axis_name="c", subcore_axis_name="s")
    cp = pltpu.CompilerParams()
    if "needs_layout_passes" in pltpu.CompilerParams.__dataclass_fields__:
        cp = dataclasses.replace(cp, needs_layout_passes=False)

    @pl.kernel(
        mesh=mesh,
        compiler_params=cp,
        out_type=jax.ShapeDtypeStruct((NUM_FIELDS, EMB, B), jnp.float32),
        scratch_types=[
            pltpu.VMEM((IDX_PER_W,), jnp.int32),
            pltpu.VMEM((IDX_PER_W,), jnp.int32),
            pltpu.VMEM((CHUNK, 128), jnp.float32),
            pltpu.VMEM((CHUNK, 128), jnp.float32),
            pltpu.VMEM((EMB, CHUNK), jnp.float32),
            pltpu.SemaphoreType.DMA,
            pltpu.SemaphoreType.DMA,
        ],
    )
    def k(tab_hbm, idx_hbm, m_hbm, out_hbm, idx_v, m_v, rows_a, rows_b,
          sel_v, sem_a, sem_b):
        wid = lax.axis_index("s") * NC + lax.axis_index("c")
        j0 = wid * CHUNKS_PER_W
        base = wid * IDX_PER_W
        pltpu.sync_copy(idx_hbm.at[pl.ds(base, IDX_PER_W)], idx_v)
        pltpu.sync_copy(m_hbm.at[pl.ds(base, IDX_PER_W)], m_v)

        lanes = lax.iota(jnp.int32, 16)

        def select_write(jj, rows_buf):
            jglob = j0 + jj
            f = jglob // BCHUNKS
            b0 = (jglob % BCHUNKS) * CHUNK

            @pl.loop(0, CHUNK // 16)
            def _(g):
                row_idx = g * 16 + lanes
                m_vec = m_v[pl.ds(jj * CHUNK + g * 16, 16)]
                for c in range(EMB):
                    vals = plsc.load_gather(rows_buf, [row_idx, m_vec + c])
                    plsc.store_scatter(sel_v, [jnp.full((16,), c, jnp.int32),
                                               row_idx], vals)
            pltpu.sync_copy(sel_v, out_hbm.at[f, :, pl.ds(b0, CHUNK)])

        pltpu.async_copy(tab_hbm.at[idx_v.at[pl.ds(0, CHUNK)]], rows_a,
                         sem_a).wait()

        @pl.loop(0, CHUNKS_PER_W - 2, step=2)
        def _(j):
            cp = pltpu.async_copy(
                tab_hbm.at[idx_v.at[pl.ds((j + 1) * CHUNK, CHUNK)]], rows_b,
                sem_b)
            select_write(j, rows_a)
            cp.wait()
            cp2 = pltpu.async_copy(
                tab_hbm.at[idx_v.at[pl.ds((j + 2) * CHUNK, CHUNK)]], rows_a,
                sem_a)
            select_write(j + 1, rows_b)
            cp2.wait()

        j_last = CHUNKS_PER_W - 2
        cp = pltpu.async_copy(
            tab_hbm.at[idx_v.at[pl.ds((j_last + 1) * CHUNK, CHUNK)]], rows_b,
            sem_b)
        select_write(j_last, rows_a)
        cp.wait()
        select_write(j_last + 1, rows_b)

    return k(tab4, idx4, m32)


def _bn_relu_t(h, g, be):
    h = jnp.maximum(h, 0.0)
    mean = jnp.mean(h, axis=1, keepdims=True)
    var = jnp.mean(h * h, axis=1, keepdims=True) - mean * mean
    inv = g * lax.rsqrt(var + EPS)
    return h * inv + (be - mean * inv)


def _mlp_body(num_ref, emb_ref, w0n_ref, w0e_ref, b0_ref, g0_ref, be0_ref,
              w1_ref, b1_ref, g1_ref, be1_ref,
              w2_ref, b2_ref, g2_ref, be2_ref,
              wh_ref, bh_ref, out_ref):
    f32 = jnp.float32
    h = jnp.dot(w0e_ref[...], emb_ref[...], preferred_element_type=f32)
    h = h + jnp.dot(w0n_ref[...], num_ref[...], preferred_element_type=f32)
    h = _bn_relu_t(h + b0_ref[...], g0_ref[...], be0_ref[...])
    h = jnp.dot(w1_ref[...], h, preferred_element_type=f32)
    h = _bn_relu_t(h + b1_ref[...], g1_ref[...], be1_ref[...])
    h = jnp.dot(w2_ref[...], h, preferred_element_type=f32)
    h = _bn_relu_t(h + b2_ref[...], g2_ref[...], be2_ref[...])
    out_ref[...] = (jnp.dot(wh_ref[...], h, preferred_element_type=f32)
                    + bh_ref[...])


def kernel(numeric, categorical, tables, W0, b0, g0, be0, W1, b1, g1, be1,
           W2, b2, g2, be2, Wh, bh):
    tab_t = tables.transpose(0, 2, 1)
    tab4 = _pack_tables(tab_t).reshape(NUM_FIELDS * VROWS, 128)

    cat_f = categorical.T
    offs = (jnp.arange(NUM_FIELDS, dtype=jnp.int32) * VROWS)[:, None]
    idx4 = ((cat_f % QUART) + offs).reshape(-1)
    m32 = ((cat_f // QUART) * 32).reshape(-1)

    emb_t = _sc_gather(tab4, idx4, m32).reshape(NUM_FIELDS * EMB, B)

    num_t = jnp.pad(numeric, ((0, 0), (0, 3))).T
    w0n = jnp.pad(W0[:, :NUM_NUMERIC], ((0, 0), (0, 3)))
    w0e = W0[:, NUM_NUMERIC:]

    out = pl.pallas_call(
        _mlp_body,
        out_shape=jax.ShapeDtypeStruct((1, B), jnp.float32),
    )(num_t, emb_t, w0n, w0e,
      b0[:, None], g0[:, None], be0[:, None],
      W1, b1[:, None], g1[:, None], be1[:, None],
      W2, b2[:, None], g2[:, None], be2[:, None],
      Wh, bh[:, None])
    return out.reshape(B)


# speedup vs baseline: 1.4227x; 1.4227x over previous
import dataclasses

import jax
import jax.numpy as jnp
from jax import lax
from jax.experimental import pallas as pl
from jax.experimental.pallas import tpu as pltpu
from jax.experimental.pallas import tpu_sc as plsc

B = 4096
NUM_NUMERIC = 13
NUM_FIELDS = 26
VOCAB = 100000
EMB = 32
EPS = 1e-5

NC, NS = 2, 16
NW = NC * NS
CHUNK = 128
BCHUNKS = B // CHUNK
TOTAL_CHUNKS = NUM_FIELDS * BCHUNKS
CHUNKS_PER_W = TOTAL_CHUNKS // NW
IDX_PER_W = CHUNKS_PER_W * CHUNK

PACK_C = 1280
PACK_NB = 20
QUART = PACK_C * PACK_NB
VROWS = QUART
LAST_COL_BLOCK = (VOCAB - 1) // PACK_C


def _pack_body(t0_ref, t1_ref, t2_ref, t3_ref, o_ref):
    c_id = pl.program_id(1)
    parts = []
    for m, t_ref in enumerate((t0_ref, t1_ref, t2_ref, t3_ref)):
        t = t_ref[0]
        if m == 3:
            src0 = jnp.minimum(m * PACK_NB + c_id, LAST_COL_BLOCK) * PACK_C
            src = src0 + lax.broadcasted_iota(jnp.int32, (EMB, PACK_C), 1)
            t = jnp.where(src < VOCAB, t, 0.0)
        parts.append(t)
    stack = jnp.concatenate(parts, axis=0)
    rows = lax.broadcasted_iota(jnp.int32, (128, 128), 0)
    cols = lax.broadcasted_iota(jnp.int32, (128, 128), 1)
    eye = (rows == cols).astype(jnp.float32)
    o_ref[0] = lax.dot_general(stack, eye, (((0,), (0,)), ((), ())),
                               preferred_element_type=jnp.float32)


def _pack_tables(tab_t):
    def spec(m):
        return pl.BlockSpec(
            (1, EMB, PACK_C),
            lambda f, c, m=m: (f, 0, jnp.minimum(m * PACK_NB + c,
                                                 LAST_COL_BLOCK)))
    return pl.pallas_call(
        _pack_body,
        grid=(NUM_FIELDS, PACK_NB),
        in_specs=[spec(0), spec(1), spec(2), spec(3)],
        out_specs=pl.BlockSpec((1, PACK_C, 128), lambda f, c: (f, c, 0)),
        out_shape=jax.ShapeDtypeStruct((NUM_FIELDS, VROWS, 128), jnp.float32),
    )(tab_t, tab_t, tab_t, tab_t)


def _sc_gather(tab4, idx4, m32):
    mesh = plsc.VectorSubcoreMesh(core_axis_name="c", subcore_axis_name="s")
    cp = pltpu.CompilerParams()
    if "needs_layout_passes" in pltpu.CompilerParams.__dataclass_fields__:
        cp = dataclasses.replace(cp, needs_layout_passes=False)

    @pl.kernel(
        mesh=mesh,
        compiler_params=cp,
        out_type=jax.ShapeDtypeStruct((NUM_FIELDS, EMB, B), jnp.float32),
        scratch_types=[
            pltpu.VMEM((IDX_PER_W,), jnp.int32),
            pltpu.VMEM((IDX_PER_W,), jnp.int32),
            pltpu.VMEM((CHUNK, 128), jnp.float32),
            pltpu.VMEM((CHUNK, 128), jnp.float32),
            pltpu.VMEM((EMB, CHUNK), jnp.float32),
            pltpu.SemaphoreType.DMA,
            pltpu.SemaphoreType.DMA,
        ],
    )
    def k(tab_hbm, idx_hbm, m_hbm, out_hbm, idx_v, m_v, rows_a, rows_b,
          sel_v, sem_a, sem_b):
        wid = lax.axis_index("s") * NC + lax.axis_index("c")
        j0 = wid * CHUNKS_PER_W
        base = wid * IDX_PER_W
        pltpu.sync_copy(idx_hbm.at[pl.ds(base, IDX_PER_W)], idx_v)
        pltpu.sync_copy(m_hbm.at[pl.ds(base, IDX_PER_W)], m_v)

        lanes = lax.iota(jnp.int32, 16)

        def select_write(jj, rows_buf):
            jglob = j0 + jj
            f = jglob // BCHUNKS
            b0 = (jglob % BCHUNKS) * CHUNK

            @pl.loop(0, CHUNK // 16)
            def _(g):
                row_idx = g * 16 + lanes
                m_vec = m_v[pl.ds(jj * CHUNK + g * 16, 16)]
                for c in range(EMB):
                    vals = plsc.load_gather(rows_buf, [row_idx, m_vec + c])
                    plsc.store_scatter(sel_v, [jnp.full((16,), c, jnp.int32),
                                               row_idx], vals)
            pltpu.sync_copy(sel_v, out_hbm.at[f, :, pl.ds(b0, CHUNK)])

        pltpu.async_copy(tab_hbm.at[idx_v.at[pl.ds(0, CHUNK)]], rows_a,
                         sem_a).wait()

        @pl.loop(0, CHUNKS_PER_W - 2, step=2)
        def _(j):
            cp = pltpu.async_copy(
                tab_hbm.at[idx_v.at[pl.ds((j + 1) * CHUNK, CHUNK)]], rows_b,
                sem_b)
            select_write(j, rows_a)
            cp.wait()
            cp2 = pltpu.async_copy(
                tab_hbm.at[idx_v.at[pl.ds((j + 2) * CHUNK, CHUNK)]], rows_a,
                sem_a)
            select_write(j + 1, rows_b)
            cp2.wait()

        j_last = CHUNKS_PER_W - 2
        cp = pltpu.async_copy(
            tab_hbm.at[idx_v.at[pl.ds((j_last + 1) * CHUNK, CHUNK)]], rows_b,
            sem_b)
        select_write(j_last, rows_a)
        cp.wait()
        select_write(j_last + 1, rows_b)

    return k(tab4, idx4, m32)


def _bn_relu_t(h, g, be):
    h = jnp.maximum(h, 0.0)
    mean = jnp.mean(h, axis=1, keepdims=True)
    var = jnp.mean(h * h, axis=1, keepdims=True) - mean * mean
    inv = g * lax.rsqrt(var + EPS)
    return h * inv + (be - mean * inv)


def _mlp_body(num_ref, emb_ref, w0n_ref, w0e_ref, b0_ref, g0_ref, be0_ref,
              w1_ref, b1_ref, g1_ref, be1_ref,
              w2_ref, b2_ref, g2_ref, be2_ref,
              wh_ref, bh_ref, out_ref):
    f32 = jnp.float32
    h = jnp.dot(w0e_ref[...], emb_ref[...], preferred_element_type=f32)
    h = h + jnp.dot(w0n_ref[...], num_ref[...], preferred_element_type=f32)
    h = _bn_relu_t(h + b0_ref[...], g0_ref[...], be0_ref[...])
    h = jnp.dot(w1_ref[...], h, preferred_element_type=f32)
    h = _bn_relu_t(h + b1_ref[...], g1_ref[...], be1_ref[...])
    h = jnp.dot(w2_ref[...], h, preferred_element_type=f32)
    h = _bn_relu_t(h + b2_ref[...], g2_ref[...], be2_ref[...])
    out_ref[...] = (jnp.dot(wh_ref[...], h, preferred_element_type=f32)
                    + bh_ref[...])


def kernel(numeric, categorical, tables, W0, b0, g0, be0, W1, b1, g1, be1,
           W2, b2, g2, be2, Wh, bh):
    tab_t = tables.transpose(0, 2, 1)
    tab4 = _pack_tables(tab_t).reshape(NUM_FIELDS * VROWS, 128)

    cat_f = categorical.T
    offs = (jnp.arange(NUM_FIELDS, dtype=jnp.int32) * VROWS)[:, None]
    idx4 = ((cat_f % QUART) + offs).reshape(-1)
    m32 = ((cat_f // QUART) * 32).reshape(-1)

    emb_t = _sc_gather(tab4, idx4, m32).reshape(NUM_FIELDS * EMB, B)

    num_t = jnp.pad(numeric, ((0, 0), (0, 3))).T
    w0n = jnp.pad(W0[:, :NUM_NUMERIC], ((0, 0), (0, 3)))
    w0e = W0[:, NUM_NUMERIC:]

    out = pl.pallas_call(
        _mlp_body,
        out_shape=jax.ShapeDtypeStruct((1, B), jnp.float32),
    )(num_t, emb_t, w0n, w0e,
      b0[:, None], g0[:, None], be0[:, None],
      W1, b1[:, None], g1[:, None], be1[:, None],
      W2, b2[:, None], g2[:, None], be2[:, None],
      Wh, bh[:, None])
    return out.reshape(B)
